# baseline (speedup 1.0000x reference)
import functools

import jax
import jax.numpy as jnp
from jax import lax
from jax.experimental import pallas as pl
from jax.experimental.pallas import tpu as pltpu
from jax.experimental.pallas import tpu_sc as plsc

N = 10000
E = 320000
D = 128
H = 64
B = 32

NPAD = 10240
NC = 2
NS = 16
NW = NC * NS
K1 = 80
C1 = 125
NBUF = 3
DBUF = 5
RPT = NPAD // NS

PERM = []
for _g in range(H // 32):
    for _k in range(16):
        PERM.append(32 * _g + _k)
        PERM.append(32 * _g + 16 + _k)


def _sc_mesh():
    return plsc.VectorSubcoreMesh(core_axis_name="c", subcore_axis_name="s")


_SC_PARAMS = pltpu.CompilerParams(use_tc_tiling_on_sc=False,
                                  needs_layout_passes=False)


def _deg_kernel(dst3, ew3):
    @functools.partial(
        pl.kernel,
        out_type=jax.ShapeDtypeStruct((NC, NPAD), jnp.float32),
        mesh=_sc_mesh(),
        compiler_params=_SC_PARAMS,
        scratch_types=(
            [pltpu.VMEM((K1,), jnp.int32)] * DBUF
            + [pltpu.VMEM((K1,), jnp.float32)] * DBUF
            + [pltpu.VMEM((RPT,), jnp.float32),
               pltpu.VMEM_SHARED((NPAD,), jnp.float32)]
            + [pltpu.SemaphoreType.DMA] * (2 * DBUF)
        ),
    )
    def k(dst_h, ew_h, out_h, d0, d1, d2, d3, d4, e0, e1, e2, e3, e4,
          zbuf, acc, *sems):
        dv = [d0, d1, d2, d3, d4]
        ev = [e0, e1, e2, e3, e4]
        sm = list(sems[:DBUF])
        ss = list(sems[DBUF:])
        cid = lax.axis_index("c")
        sid = lax.axis_index("s")
        wid = cid * NS + sid

        for i in range(RPT // 16):
            zbuf[pl.ds(i * 16, 16)] = jnp.zeros((16,), jnp.float32)
        pltpu.sync_copy(zbuf, acc.at[pl.ds(sid * RPT, RPT)])
        plsc.subcore_barrier()

        def issue_meta(c, s):
            pltpu.async_copy(dst_h.at[wid, c], dv[s], sm[s])
            pltpu.async_copy(ew_h.at[wid, c], ev[s], sm[s])

        def wait_meta(c, s):
            pltpu.make_async_copy(dst_h.at[wid, c], dv[s], sm[s]).wait()
            pltpu.make_async_copy(ew_h.at[wid, c], ev[s], sm[s]).wait()

        for c0 in range(3):
            issue_meta(c0, c0)

        def emit(c, b, sdescs, prefetch):
            wait_meta(c, b)
            sdescs[b] = pltpu.async_copy(ev[b], acc.at[dv[b]], ss[b],
                                         add=True)
            if b >= 2:
                sdescs[b - 2].wait()
            if prefetch:
                issue_meta(c + 3, (b + 3) % DBUF)
            return sdescs

        def body(gi, carry):
            sdescs = [None] * DBUF
            for b in range(DBUF):
                sdescs = emit(gi * DBUF + b, b, sdescs, True)
            sdescs[DBUF - 2].wait()
            sdescs[DBUF - 1].wait()
            return carry

        lax.fori_loop(0, (C1 // DBUF) - 1, body, 0)
        sdescs = [None] * DBUF
        for b in range(DBUF):
            c = C1 - DBUF + b
            sdescs = emit(c, b, sdescs, c + 3 < C1)
        sdescs[DBUF - 2].wait()
        sdescs[DBUF - 1].wait()

        plsc.subcore_barrier()
        pltpu.sync_copy(acc.at[pl.ds(sid * RPT, RPT)],
                        out_h.at[cid, pl.ds(sid * RPT, RPT)])

    return k(dst3, ew3)


def _agg_kernel(g, src3, dst3, ew3):
    @functools.partial(
        pl.kernel,
        out_type=jax.ShapeDtypeStruct((NC, NPAD, H), jnp.float32),
        mesh=_sc_mesh(),
        compiler_params=_SC_PARAMS,
        scratch_types=(
            [pltpu.VMEM((K1,), jnp.int32)] * NBUF
            + [pltpu.VMEM((K1,), jnp.int32)] * NBUF
            + [pltpu.VMEM((K1,), jnp.float32)] * NBUF
            + [pltpu.VMEM((K1, H), jnp.bfloat16)] * NBUF
            + [pltpu.VMEM((K1, H), jnp.float32)] * NBUF
            + [pltpu.VMEM((64, H), jnp.float32),
               pltpu.VMEM_SHARED((NPAD, H), jnp.float32)]
            + [pltpu.SemaphoreType.DMA] * (3 * NBUF)
        ),
    )
    def k(g_h, src_h, dst_h, ew_h, out_h,
          sv0, sv1, sv2, dv0, dv1, dv2, ev0, ev1, ev2,
          r0, r1, r2, f0, f1, f2, zbuf, acc, *sems):
        sv = [sv0, sv1, sv2]
        dv = [dv0, dv1, dv2]
        ev = [ev0, ev1, ev2]
        rows = [r0, r1, r2]
        fb = [f0, f1, f2]
        sm = list(sems[:NBUF])
        sg = list(sems[NBUF:2 * NBUF])
        ss = list(sems[2 * NBUF:])
        cid = lax.axis_index("c")
        sid = lax.axis_index("s")
        wid = cid * NS + sid

        def zfill(i, carry):
            for j in range(H // 16):
                zbuf[i, pl.ds(j * 16, 16)] = jnp.zeros((16,), jnp.float32)
            return carry

        lax.fori_loop(0, 64, zfill, 0)

        def zcopy(i, carry):
            pltpu.sync_copy(zbuf, acc.at[pl.ds(sid * RPT + i * 64, 64)])
            return carry

        lax.fori_loop(0, RPT // 64, zcopy, 0)
        plsc.subcore_barrier()

        def issue_meta(c, s):
            pltpu.async_copy(src_h.at[wid, c], sv[s], sm[s])
            pltpu.async_copy(dst_h.at[wid, c], dv[s], sm[s])
            pltpu.async_copy(ew_h.at[wid, c], ev[s], sm[s])

        def wait_meta(c, s):
            pltpu.make_async_copy(src_h.at[wid, c], sv[s], sm[s]).wait()
            pltpu.make_async_copy(dst_h.at[wid, c], dv[s], sm[s]).wait()
            pltpu.make_async_copy(ew_h.at[wid, c], ev[s], sm[s]).wait()

        issue_meta(0, 0)
        issue_meta(1, 1)
        wait_meta(0, 0)
        pltpu.async_copy(g_h.at[sv[0]], rows[0], sg[0])

        def emit(c, b, sdescs, pf1, pf2):
            s, s1, s2 = b, (b + 1) % NBUF, (b + 2) % NBUF
            pltpu.make_async_copy(g_h.at[sv[s]], rows[s], sg[s]).wait()
            if pf1:
                wait_meta(c + 1, s1)
                pltpu.async_copy(g_h.at[sv[s1]], rows[s1], sg[s1])
            rs = rows[s]
            fs = fb[s]

            def scale(gg, c2):
                ew16 = ev[s][pl.ds(gg * 16, 16)]
                for j in range(16):
                    e = gg * 16 + j
                    w = ew16[j]
                    for f in range(H // 32):
                        x32 = rs[e, pl.ds(f * 32, 32)]
                        lo, hi = plsc.unpack(
                            x32, format=plsc.PackFormat.INTERLEAVED)
                        fs[e, pl.ds(f * 32, 16)] = lo * w
                        fs[e, pl.ds(f * 32 + 16, 16)] = hi * w
                return c2

            lax.fori_loop(0, K1 // 16, scale, 0)

            sdescs[b] = pltpu.async_copy(fb[s], acc.at[dv[s]], ss[s],
                                         add=True)
            if b >= 1:
                sdescs[b - 1].wait()
            if pf2:
                issue_meta(c + 2, s2)
            return sdescs

        def body(gi, carry):
            sdescs = [None] * NBUF
            for b in range(NBUF):
                sdescs = emit(gi * NBUF + b, b, sdescs, True, True)
            sdescs[NBUF - 1].wait()
            return carry

        lax.fori_loop(0, (C1 - 2) // NBUF, body, 0)
        sdescs = [None] * NBUF
        sdescs = emit(C1 - 2, 0, sdescs, True, False)
        sdescs = emit(C1 - 1, 1, sdescs, False, False)
        sdescs[1].wait()

        plsc.subcore_barrier()
        pltpu.sync_copy(acc.at[pl.ds(sid * RPT, RPT)],
                        out_h.at[cid, pl.ds(sid * RPT, RPT)])

    return k(g, src3, dst3, ew3)


def _tc1a(x, W1, W1p):
    def body(x_ref, w_ref, wp_ref, h_ref, hp_ref):
        x = x_ref[...]
        zp = jnp.zeros((NPAD - N, H), jnp.float32)
        h_ref[...] = jnp.concatenate(
            [jnp.dot(x, w_ref[...], preferred_element_type=jnp.float32), zp],
            axis=0)
        hp_ref[...] = jnp.concatenate(
            [jnp.dot(x, wp_ref[...], preferred_element_type=jnp.float32), zp],
            axis=0)

    return pl.pallas_call(
        body,
        out_shape=(
            jax.ShapeDtypeStruct((NPAD, H), jnp.float32),
            jax.ShapeDtypeStruct((NPAD, H), jnp.float32),
        ),
    )(x, W1, W1p)


def _tc1b(h, hp, degp):
    def body(h_ref, hp_ref, degp_ref, g_ref, gb_ref, dis_ref):
        deg = degp_ref[0] + degp_ref[1] + 1.0
        dis = lax.rsqrt(deg)
        g_ref[...] = h_ref[...] * dis
        gb_ref[...] = (hp_ref[...] * dis).astype(jnp.bfloat16)
        dis_ref[...] = dis

    return pl.pallas_call(
        body,
        out_shape=(
            jax.ShapeDtypeStruct((NPAD, H), jnp.float32),
            jax.ShapeDtypeStruct((NPAD, H), jnp.bfloat16),
            jax.ShapeDtypeStruct((NPAD, 1), jnp.float32),
        ),
    )(h, hp, degp)


def _tc2(aggp, g1, dis, W2, W2p, b1row):
    def body(aggp_ref, g1_ref, dis_ref, w_ref, wp_ref, b_ref, g2_ref, gb_ref):
        dis = dis_ref[...]
        h = jnp.maximum(
            (aggp_ref[0] + aggp_ref[1] + g1_ref[...]) * dis + b_ref[...], 0.0)
        t = jnp.dot(h, w_ref[...], preferred_element_type=jnp.float32)
        tp = jnp.dot(h, wp_ref[...], preferred_element_type=jnp.float32)
        g2_ref[...] = t * dis
        gb_ref[...] = (tp * dis).astype(jnp.bfloat16)

    return pl.pallas_call(
        body,
        out_shape=(
            jax.ShapeDtypeStruct((NPAD, H), jnp.float32),
            jax.ShapeDtypeStruct((NPAD, H), jnp.bfloat16),
        ),
    )(aggp, g1, dis, W2, W2p, b1row)


def _tc3(aggp, g2, dis, b2row, brow, M1, mb1row, M2p, mb2row):
    def body(aggp_ref, g2_ref, dis_ref, b2_ref, brow_ref, m1_ref, mb1_ref,
             m2_ref, mb2_ref, out_ref):
        dis = dis_ref[...]
        h2 = jnp.maximum(
            (aggp_ref[0] + aggp_ref[1] + g2_ref[...]) * dis + b2_ref[...], 0.0)
        gids = lax.broadcasted_iota(jnp.int32, (B, NPAD), 0)
        oh = (brow_ref[...] == gids).astype(jnp.float32)
        ssum = jnp.dot(oh, h2, preferred_element_type=jnp.float32)
        cnt = jnp.dot(oh, jnp.ones((NPAD, 1), jnp.float32),
                      preferred_element_type=jnp.float32)
        pooled = ssum / jnp.maximum(cnt, 1.0)
        z = jnp.maximum(
            jnp.dot(pooled, m1_ref[...], preferred_element_type=jnp.float32)
            + mb1_ref[...], 0.0)
        out_ref[...] = (
            jnp.dot(z, m2_ref[...], preferred_element_type=jnp.float32)
            + mb2_ref[...])

    return pl.pallas_call(
        body,
        out_shape=jax.ShapeDtypeStruct((B, 128), jnp.float32),
    )(aggp, g2, dis, b2row, brow, M1, mb1row, M2p, mb2row)


@jax.jit
def kernel(x, ei, ew, b, W1, b1, W2, b2, M1, mb1, M2, mb2):
    src3 = ei[0].reshape(NW, C1, K1)
    dst3 = ei[1].reshape(NW, C1, K1)
    ew3 = ew.reshape(NW, C1, K1)
    brow = jnp.pad(b, (0, NPAD - N), constant_values=B)[None, :]
    b1row = b1[None, :]
    b2row = b2[None, :]
    mb1row = mb1[None, :]
    M2p = jnp.pad(M2, ((0, 0), (0, 128 - M2.shape[1])))
    mb2row = jnp.pad(mb2, (0, 128 - mb2.shape[0]))[None, :]
    perm = jnp.asarray(PERM, dtype=jnp.int32)
    W1p = jnp.take(W1, perm, axis=1)
    W2p = jnp.take(W2, perm, axis=1)

    h, hp = _tc1a(x, W1, W1p)
    degp = _deg_kernel(dst3, ew3)
    degp3 = degp[:, :, None]

    g1, gb1, dis = _tc1b(h, hp, degp3)
    agg1 = _agg_kernel(gb1, src3, dst3, ew3)
    g2, gb2 = _tc2(agg1, g1, dis, W2, W2p, b1row)
    agg2 = _agg_kernel(gb2, src3, dst3, ew3)
    out_full = _tc3(agg2, g2, dis, b2row, brow, M1, mb1row, M2p, mb2row)
    return out_full[:, :1]


# speedup vs baseline: 1.0368x; 1.0368x over previous
import functools

import jax
import jax.numpy as jnp
from jax import lax
from jax.experimental import pallas as pl
from jax.experimental.pallas import tpu as pltpu
from jax.experimental.pallas import tpu_sc as plsc

N = 10000
E = 320000
D = 128
H = 64
B = 32

NPAD = 10240
NC = 2
NS = 16
NW = NC * NS
K1 = 80
C1 = 125
NBUF = 3
DBUF = 5
RPT = NPAD // NS

PERM = []
for _g in range(H // 32):
    for _k in range(16):
        PERM.append(32 * _g + _k)
        PERM.append(32 * _g + 16 + _k)


def _sc_mesh():
    return plsc.VectorSubcoreMesh(core_axis_name="c", subcore_axis_name="s")


_SC_PARAMS = pltpu.CompilerParams(use_tc_tiling_on_sc=False,
                                  needs_layout_passes=False)


def _deg_kernel(dst3, ew3):
    @functools.partial(
        pl.kernel,
        out_type=jax.ShapeDtypeStruct((NC, NPAD), jnp.float32),
        mesh=_sc_mesh(),
        compiler_params=_SC_PARAMS,
        scratch_types=(
            [pltpu.VMEM((K1,), jnp.int32)] * DBUF
            + [pltpu.VMEM((K1,), jnp.float32)] * DBUF
            + [pltpu.VMEM((RPT,), jnp.float32),
               pltpu.VMEM_SHARED((NPAD,), jnp.float32)]
            + [pltpu.SemaphoreType.DMA] * (2 * DBUF)
        ),
    )
    def k(dst_h, ew_h, out_h, d0, d1, d2, d3, d4, e0, e1, e2, e3, e4,
          zbuf, acc, *sems):
        dv = [d0, d1, d2, d3, d4]
        ev = [e0, e1, e2, e3, e4]
        sm = list(sems[:DBUF])
        ss = list(sems[DBUF:])
        cid = lax.axis_index("c")
        sid = lax.axis_index("s")
        wid = cid * NS + sid

        for i in range(RPT // 16):
            zbuf[pl.ds(i * 16, 16)] = jnp.zeros((16,), jnp.float32)
        pltpu.sync_copy(zbuf, acc.at[pl.ds(sid * RPT, RPT)])
        plsc.subcore_barrier()

        def issue_meta(c, s):
            pltpu.async_copy(dst_h.at[wid, c], dv[s], sm[s])
            pltpu.async_copy(ew_h.at[wid, c], ev[s], sm[s])

        def wait_meta(c, s):
            pltpu.make_async_copy(dst_h.at[wid, c], dv[s], sm[s]).wait()
            pltpu.make_async_copy(ew_h.at[wid, c], ev[s], sm[s]).wait()

        for c0 in range(3):
            issue_meta(c0, c0)

        def emit(c, b, sdescs, prefetch):
            wait_meta(c, b)
            sdescs[b] = pltpu.async_copy(ev[b], acc.at[dv[b]], ss[b],
                                         add=True)
            if b >= 2:
                sdescs[b - 2].wait()
            if prefetch:
                issue_meta(c + 3, (b + 3) % DBUF)
            return sdescs

        def body(gi, carry):
            sdescs = [None] * DBUF
            for b in range(DBUF):
                sdescs = emit(gi * DBUF + b, b, sdescs, True)
            sdescs[DBUF - 2].wait()
            sdescs[DBUF - 1].wait()
            return carry

        lax.fori_loop(0, (C1 // DBUF) - 1, body, 0)
        sdescs = [None] * DBUF
        for b in range(DBUF):
            c = C1 - DBUF + b
            sdescs = emit(c, b, sdescs, c + 3 < C1)
        sdescs[DBUF - 2].wait()
        sdescs[DBUF - 1].wait()

        plsc.subcore_barrier()
        pltpu.sync_copy(acc.at[pl.ds(sid * RPT, RPT)],
                        out_h.at[cid, pl.ds(sid * RPT, RPT)])

    return k(dst3, ew3)


def _agg_kernel(g, src3, dst3, ew3):
    @functools.partial(
        pl.kernel,
        out_type=jax.ShapeDtypeStruct((NC, NPAD, H), jnp.float32),
        mesh=_sc_mesh(),
        compiler_params=_SC_PARAMS,
        scratch_types=(
            [pltpu.VMEM((K1,), jnp.int32)] * NBUF
            + [pltpu.VMEM((K1,), jnp.int32)] * NBUF
            + [pltpu.VMEM((K1,), jnp.float32)] * NBUF
            + [pltpu.VMEM((K1, H), jnp.bfloat16)] * NBUF
            + [pltpu.VMEM((K1, H), jnp.float32)] * NBUF
            + [pltpu.VMEM((64, H), jnp.float32),
               pltpu.VMEM_SHARED((NPAD, H), jnp.float32)]
            + [pltpu.SemaphoreType.DMA] * (3 * NBUF)
        ),
    )
    def k(g_h, src_h, dst_h, ew_h, out_h,
          sv0, sv1, sv2, dv0, dv1, dv2, ev0, ev1, ev2,
          r0, r1, r2, f0, f1, f2, zbuf, acc, *sems):
        sv = [sv0, sv1, sv2]
        dv = [dv0, dv1, dv2]
        ev = [ev0, ev1, ev2]
        rows = [r0, r1, r2]
        fb = [f0, f1, f2]
        sm = list(sems[:NBUF])
        sg = list(sems[NBUF:2 * NBUF])
        ss = list(sems[2 * NBUF:])
        cid = lax.axis_index("c")
        sid = lax.axis_index("s")
        wid = cid * NS + sid

        def zfill(i, carry):
            for j in range(H // 16):
                zbuf[i, pl.ds(j * 16, 16)] = jnp.zeros((16,), jnp.float32)
            return carry

        lax.fori_loop(0, 64, zfill, 0)

        def zcopy(i, carry):
            pltpu.sync_copy(zbuf, acc.at[pl.ds(sid * RPT + i * 64, 64)])
            return carry

        lax.fori_loop(0, RPT // 64, zcopy, 0)
        plsc.subcore_barrier()

        def issue_meta(c, s):
            pltpu.async_copy(src_h.at[wid, c], sv[s], sm[s])
            pltpu.async_copy(dst_h.at[wid, c], dv[s], sm[s])
            pltpu.async_copy(ew_h.at[wid, c], ev[s], sm[s])

        def wait_meta(c, s):
            pltpu.make_async_copy(src_h.at[wid, c], sv[s], sm[s]).wait()
            pltpu.make_async_copy(dst_h.at[wid, c], dv[s], sm[s]).wait()
            pltpu.make_async_copy(ew_h.at[wid, c], ev[s], sm[s]).wait()

        issue_meta(0, 0)
        issue_meta(1, 1)
        wait_meta(0, 0)
        pltpu.async_copy(g_h.at[sv[0]], rows[0], sg[0])

        def emit(c, b, sdescs, pf1, pf2):
            s, s1, s2 = b, (b + 1) % NBUF, (b + 2) % NBUF
            pltpu.make_async_copy(g_h.at[sv[s]], rows[s], sg[s]).wait()
            if pf1:
                wait_meta(c + 1, s1)
                pltpu.async_copy(g_h.at[sv[s1]], rows[s1], sg[s1])
            rs = rows[s]
            fs = fb[s]

            def scale(gg, c2):
                ew16 = ev[s][pl.ds(gg * 16, 16)]
                for j in range(16):
                    e = gg * 16 + j
                    w = ew16[j]
                    for f in range(H // 32):
                        x32 = rs[e, pl.ds(f * 32, 32)]
                        lo, hi = plsc.unpack(
                            x32, format=plsc.PackFormat.INTERLEAVED)
                        fs[e, pl.ds(f * 32, 16)] = lo * w
                        fs[e, pl.ds(f * 32 + 16, 16)] = hi * w
                return c2

            lax.fori_loop(0, K1 // 16, scale, 0)

            sdescs[b] = pltpu.async_copy(fb[s], acc.at[dv[s]], ss[s],
                                         add=True)
            if b >= 1:
                sdescs[b - 1].wait()
            if pf2:
                issue_meta(c + 2, s2)
            return sdescs

        def body(gi, carry):
            sdescs = [None] * NBUF
            for b in range(NBUF):
                sdescs = emit(gi * NBUF + b, b, sdescs, True, True)
            sdescs[NBUF - 1].wait()
            return carry

        lax.fori_loop(0, (C1 - 2) // NBUF, body, 0)
        sdescs = [None] * NBUF
        sdescs = emit(C1 - 2, 0, sdescs, True, False)
        sdescs = emit(C1 - 1, 1, sdescs, False, False)
        sdescs[1].wait()

        plsc.subcore_barrier()
        pltpu.sync_copy(acc.at[pl.ds(sid * RPT, RPT)],
                        out_h.at[cid, pl.ds(sid * RPT, RPT)])

    return k(g, src3, dst3, ew3)


def _tc1a(x, W1, W1p):
    def body(x_ref, w_ref, wp_ref, h_ref, hp_ref):
        x = x_ref[...]
        zp = jnp.zeros((NPAD - N, H), jnp.float32)
        h_ref[...] = jnp.concatenate(
            [jnp.dot(x, w_ref[...], preferred_element_type=jnp.float32), zp],
            axis=0)
        hp_ref[...] = jnp.concatenate(
            [jnp.dot(x, wp_ref[...], preferred_element_type=jnp.float32), zp],
            axis=0)

    return pl.pallas_call(
        body,
        out_shape=(
            jax.ShapeDtypeStruct((NPAD, H), jnp.float32),
            jax.ShapeDtypeStruct((NPAD, H), jnp.float32),
        ),
    )(x, W1, W1p)


def _dis_col(degp_ref):
    deg = degp_ref[0] + degp_ref[1] + 1.0
    return lax.rsqrt(deg).reshape(NPAD, 1)


def _tc1b(h, hp, degp):
    def body(h_ref, hp_ref, degp_ref, g_ref, gb_ref):
        dis = _dis_col(degp_ref)
        g_ref[...] = h_ref[...] * dis
        gb_ref[...] = (hp_ref[...] * dis).astype(jnp.bfloat16)

    return pl.pallas_call(
        body,
        out_shape=(
            jax.ShapeDtypeStruct((NPAD, H), jnp.float32),
            jax.ShapeDtypeStruct((NPAD, H), jnp.bfloat16),
        ),
    )(h, hp, degp)


def _tc2(aggp, g1, degp, W2, W2p, b1row):
    def body(aggp_ref, g1_ref, degp_ref, w_ref, wp_ref, b_ref, g2_ref, gb_ref):
        dis = _dis_col(degp_ref)
        h = jnp.maximum(
            (aggp_ref[0] + aggp_ref[1] + g1_ref[...]) * dis + b_ref[...], 0.0)
        t = jnp.dot(h, w_ref[...], preferred_element_type=jnp.float32)
        tp = jnp.dot(h, wp_ref[...], preferred_element_type=jnp.float32)
        g2_ref[...] = t * dis
        gb_ref[...] = (tp * dis).astype(jnp.bfloat16)

    return pl.pallas_call(
        body,
        out_shape=(
            jax.ShapeDtypeStruct((NPAD, H), jnp.float32),
            jax.ShapeDtypeStruct((NPAD, H), jnp.bfloat16),
        ),
    )(aggp, g1, degp, W2, W2p, b1row)


def _tc3(aggp, g2, degp, b2row, brow, M1, mb1row, M2p, mb2row):
    def body(aggp_ref, g2_ref, degp_ref, b2_ref, brow_ref, m1_ref, mb1_ref,
             m2_ref, mb2_ref, out_ref):
        dis = _dis_col(degp_ref)
        h2 = jnp.maximum(
            (aggp_ref[0] + aggp_ref[1] + g2_ref[...]) * dis + b2_ref[...], 0.0)
        gids = lax.broadcasted_iota(jnp.int32, (B, NPAD), 0)
        oh = (brow_ref[...] == gids).astype(jnp.float32)
        ssum = jnp.dot(oh, h2, preferred_element_type=jnp.float32)
        cnt = jnp.dot(oh, jnp.ones((NPAD, 1), jnp.float32),
                      preferred_element_type=jnp.float32)
        pooled = ssum / jnp.maximum(cnt, 1.0)
        z = jnp.maximum(
            jnp.dot(pooled, m1_ref[...], preferred_element_type=jnp.float32)
            + mb1_ref[...], 0.0)
        out_ref[...] = (
            jnp.dot(z, m2_ref[...], preferred_element_type=jnp.float32)
            + mb2_ref[...])

    return pl.pallas_call(
        body,
        out_shape=jax.ShapeDtypeStruct((B, 128), jnp.float32),
    )(aggp, g2, degp, b2row, brow, M1, mb1row, M2p, mb2row)


@jax.jit
def kernel(x, ei, ew, b, W1, b1, W2, b2, M1, mb1, M2, mb2):
    src3 = ei[0].reshape(NW, C1, K1)
    dst3 = ei[1].reshape(NW, C1, K1)
    ew3 = ew.reshape(NW, C1, K1)
    brow = jnp.pad(b, (0, NPAD - N), constant_values=B)[None, :]
    b1row = b1[None, :]
    b2row = b2[None, :]
    mb1row = mb1[None, :]
    M2p = jnp.pad(M2, ((0, 0), (0, 128 - M2.shape[1])))
    mb2row = jnp.pad(mb2, (0, 128 - mb2.shape[0]))[None, :]
    perm = jnp.asarray(PERM, dtype=jnp.int32)
    W1p = jnp.take(W1, perm, axis=1)
    W2p = jnp.take(W2, perm, axis=1)

    h, hp = _tc1a(x, W1, W1p)
    degp = _deg_kernel(dst3, ew3)

    g1, gb1 = _tc1b(h, hp, degp)
    agg1 = _agg_kernel(gb1, src3, dst3, ew3)
    g2, gb2 = _tc2(agg1, g1, degp, W2, W2p, b1row)
    agg2 = _agg_kernel(gb2, src3, dst3, ew3)
    out_full = _tc3(agg2, g2, degp, b2row, brow, M1, mb1row, M2p, mb2row)
    return out_full[:, :1]


# speedup vs baseline: 1.0995x; 1.0605x over previous
import functools

import jax
import jax.numpy as jnp
from jax import lax
from jax.experimental import pallas as pl
from jax.experimental.pallas import tpu as pltpu
from jax.experimental.pallas import tpu_sc as plsc

N = 10000
E = 320000
D = 128
H = 64
B = 32

NPAD = 10240
NC = 2
NS = 16
NW = NC * NS
K1 = 128
C1 = 81
EPW = C1 * K1
EPAD = NW * EPW
NBUF = 3
DBUF = 5
RPT = NPAD // NS

PERM = []
for _g in range(H // 32):
    for _k in range(16):
        PERM.append(32 * _g + _k)
        PERM.append(32 * _g + 16 + _k)


def _sc_mesh():
    return plsc.VectorSubcoreMesh(core_axis_name="c", subcore_axis_name="s")


_SC_PARAMS = pltpu.CompilerParams(use_tc_tiling_on_sc=False,
                                  needs_layout_passes=False)


def _deg_kernel(dst3, ew3):
    @functools.partial(
        pl.kernel,
        out_type=jax.ShapeDtypeStruct((NC, NPAD), jnp.float32),
        mesh=_sc_mesh(),
        compiler_params=_SC_PARAMS,
        scratch_types=(
            [pltpu.VMEM((K1,), jnp.int32)] * DBUF
            + [pltpu.VMEM((K1,), jnp.float32)] * DBUF
            + [pltpu.VMEM((RPT,), jnp.float32),
               pltpu.VMEM_SHARED((NPAD,), jnp.float32)]
            + [pltpu.SemaphoreType.DMA] * (2 * DBUF)
        ),
    )
    def k(dst_h, ew_h, out_h, d0, d1, d2, d3, d4, e0, e1, e2, e3, e4,
          zbuf, acc, *sems):
        dv = [d0, d1, d2, d3, d4]
        ev = [e0, e1, e2, e3, e4]
        sm = list(sems[:DBUF])
        ss = list(sems[DBUF:])
        cid = lax.axis_index("c")
        sid = lax.axis_index("s")
        wid = cid * NS + sid

        for i in range(RPT // 16):
            zbuf[pl.ds(i * 16, 16)] = jnp.zeros((16,), jnp.float32)
        pltpu.sync_copy(zbuf, acc.at[pl.ds(sid * RPT, RPT)])
        plsc.subcore_barrier()

        def issue_meta(c, s):
            pltpu.async_copy(dst_h.at[wid, c], dv[s], sm[s])
            pltpu.async_copy(ew_h.at[wid, c], ev[s], sm[s])

        def wait_meta(c, s):
            pltpu.make_async_copy(dst_h.at[wid, c], dv[s], sm[s]).wait()
            pltpu.make_async_copy(ew_h.at[wid, c], ev[s], sm[s]).wait()

        for c0 in range(3):
            issue_meta(c0, c0)

        def emit(c, b, sdescs, prefetch):
            wait_meta(c, b)
            sdescs[b] = pltpu.async_copy(ev[b], acc.at[dv[b]], ss[b],
                                         add=True)
            if b >= 2:
                sdescs[b - 2].wait()
            if prefetch:
                issue_meta(c + 3, (b + 3) % DBUF)
            return sdescs

        def body(gi, carry):
            sdescs = [None] * DBUF
            for b in range(DBUF):
                sdescs = emit(gi * DBUF + b, b, sdescs, True)
            sdescs[DBUF - 2].wait()
            sdescs[DBUF - 1].wait()
            return carry

        ntrip = (C1 - 3) // DBUF
        lax.fori_loop(0, ntrip, body, 0)
        descs = {}
        for c in range(ntrip * DBUF, C1):
            b = c % DBUF
            wait_meta(c, b)
            descs[c] = pltpu.async_copy(ev[b], acc.at[dv[b]], ss[b],
                                        add=True)
            if c - 2 >= ntrip * DBUF:
                descs[c - 2].wait()
            if c + 3 < C1:
                issue_meta(c + 3, (c + 3) % DBUF)
        descs[C1 - 2].wait()
        descs[C1 - 1].wait()

        plsc.subcore_barrier()
        pltpu.sync_copy(acc.at[pl.ds(sid * RPT, RPT)],
                        out_h.at[cid, pl.ds(sid * RPT, RPT)])

    return k(dst3, ew3)


def _agg_kernel(g, src3, dst3, ew3):
    @functools.partial(
        pl.kernel,
        out_type=jax.ShapeDtypeStruct((NC, NPAD, H), jnp.float32),
        mesh=_sc_mesh(),
        compiler_params=_SC_PARAMS,
        scratch_types=(
            [pltpu.VMEM((K1,), jnp.int32)] * NBUF
            + [pltpu.VMEM((K1,), jnp.int32)] * NBUF
            + [pltpu.VMEM((K1,), jnp.float32)] * NBUF
            + [pltpu.VMEM((K1, H), jnp.bfloat16)] * NBUF
            + [pltpu.VMEM((K1, H), jnp.float32)] * NBUF
            + [pltpu.VMEM((64, H), jnp.float32),
               pltpu.VMEM_SHARED((NPAD, H), jnp.float32)]
            + [pltpu.SemaphoreType.DMA] * (3 * NBUF)
        ),
    )
    def k(g_h, src_h, dst_h, ew_h, out_h,
          sv0, sv1, sv2, dv0, dv1, dv2, ev0, ev1, ev2,
          r0, r1, r2, f0, f1, f2, zbuf, acc, *sems):
        sv = [sv0, sv1, sv2]
        dv = [dv0, dv1, dv2]
        ev = [ev0, ev1, ev2]
        rows = [r0, r1, r2]
        fb = [f0, f1, f2]
        sm = list(sems[:NBUF])
        sg = list(sems[NBUF:2 * NBUF])
        ss = list(sems[2 * NBUF:])
        cid = lax.axis_index("c")
        sid = lax.axis_index("s")
        wid = cid * NS + sid

        def zfill(i, carry):
            for j in range(H // 16):
                zbuf[i, pl.ds(j * 16, 16)] = jnp.zeros((16,), jnp.float32)
            return carry

        lax.fori_loop(0, 64, zfill, 0)

        def zcopy(i, carry):
            pltpu.sync_copy(zbuf, acc.at[pl.ds(sid * RPT + i * 64, 64)])
            return carry

        lax.fori_loop(0, RPT // 64, zcopy, 0)
        plsc.subcore_barrier()

        def issue_meta(c, s):
            pltpu.async_copy(src_h.at[wid, c], sv[s], sm[s])
            pltpu.async_copy(dst_h.at[wid, c], dv[s], sm[s])
            pltpu.async_copy(ew_h.at[wid, c], ev[s], sm[s])

        def wait_meta(c, s):
            pltpu.make_async_copy(src_h.at[wid, c], sv[s], sm[s]).wait()
            pltpu.make_async_copy(dst_h.at[wid, c], dv[s], sm[s]).wait()
            pltpu.make_async_copy(ew_h.at[wid, c], ev[s], sm[s]).wait()

        issue_meta(0, 0)
        issue_meta(1, 1)
        wait_meta(0, 0)
        pltpu.async_copy(g_h.at[sv[0]], rows[0], sg[0])

        def emit(c, b, sdescs, pf1, pf2):
            s, s1, s2 = b, (b + 1) % NBUF, (b + 2) % NBUF
            pltpu.make_async_copy(g_h.at[sv[s]], rows[s], sg[s]).wait()
            if pf1:
                wait_meta(c + 1, s1)
                pltpu.async_copy(g_h.at[sv[s1]], rows[s1], sg[s1])
            rs = rows[s]
            fs = fb[s]

            def scale(gg, c2):
                ew16 = ev[s][pl.ds(gg * 16, 16)]
                for j in range(16):
                    e = gg * 16 + j
                    w = ew16[j]
                    for f in range(H // 32):
                        x32 = rs[e, pl.ds(f * 32, 32)]
                        lo, hi = plsc.unpack(
                            x32, format=plsc.PackFormat.INTERLEAVED)
                        fs[e, pl.ds(f * 32, 16)] = lo * w
                        fs[e, pl.ds(f * 32 + 16, 16)] = hi * w
                return c2

            lax.fori_loop(0, K1 // 16, scale, 0)

            sdescs[b] = pltpu.async_copy(fb[s], acc.at[dv[s]], ss[s],
                                         add=True)
            if b >= 1:
                sdescs[b - 1].wait()
            if pf2:
                issue_meta(c + 2, s2)
            return sdescs

        def body(gi, carry):
            sdescs = [None] * NBUF
            for b in range(NBUF):
                sdescs = emit(gi * NBUF + b, b, sdescs, True, True)
            sdescs[NBUF - 1].wait()
            return carry

        lax.fori_loop(0, (C1 - 3) // NBUF, body, 0)
        sdescs = [None] * NBUF
        sdescs = emit(C1 - 3, 0, sdescs, True, True)
        sdescs = emit(C1 - 2, 1, sdescs, True, False)
        sdescs = emit(C1 - 1, 2, sdescs, False, False)
        sdescs[2].wait()

        plsc.subcore_barrier()
        pltpu.sync_copy(acc.at[pl.ds(sid * RPT, RPT)],
                        out_h.at[cid, pl.ds(sid * RPT, RPT)])

    return k(g, src3, dst3, ew3)


def _tc1a(x, W1, W1p):
    def body(x_ref, w_ref, wp_ref, h_ref, hp_ref):
        x = x_ref[...]
        zp = jnp.zeros((NPAD - N, H), jnp.float32)
        h_ref[...] = jnp.concatenate(
            [jnp.dot(x, w_ref[...], preferred_element_type=jnp.float32), zp],
            axis=0)
        hp_ref[...] = jnp.concatenate(
            [jnp.dot(x, wp_ref[...], preferred_element_type=jnp.float32), zp],
            axis=0)

    return pl.pallas_call(
        body,
        out_shape=(
            jax.ShapeDtypeStruct((NPAD, H), jnp.float32),
            jax.ShapeDtypeStruct((NPAD, H), jnp.float32),
        ),
    )(x, W1, W1p)


def _dis_col(degp_ref):
    deg = degp_ref[0] + degp_ref[1] + 1.0
    return lax.rsqrt(deg).reshape(NPAD, 1)


def _tc1b(h, hp, degp):
    def body(h_ref, hp_ref, degp_ref, g_ref, gb_ref):
        dis = _dis_col(degp_ref)
        g_ref[...] = h_ref[...] * dis
        gb_ref[...] = (hp_ref[...] * dis).astype(jnp.bfloat16)

    return pl.pallas_call(
        body,
        out_shape=(
            jax.ShapeDtypeStruct((NPAD, H), jnp.float32),
            jax.ShapeDtypeStruct((NPAD, H), jnp.bfloat16),
        ),
    )(h, hp, degp)


def _tc2(aggp, g1, degp, W2, W2p, b1row):
    def body(aggp_ref, g1_ref, degp_ref, w_ref, wp_ref, b_ref, g2_ref, gb_ref):
        dis = _dis_col(degp_ref)
        h = jnp.maximum(
            (aggp_ref[0] + aggp_ref[1] + g1_ref[...]) * dis + b_ref[...], 0.0)
        t = jnp.dot(h, w_ref[...], preferred_element_type=jnp.float32)
        tp = jnp.dot(h, wp_ref[...], preferred_element_type=jnp.float32)
        g2_ref[...] = t * dis
        gb_ref[...] = (tp * dis).astype(jnp.bfloat16)

    return pl.pallas_call(
        body,
        out_shape=(
            jax.ShapeDtypeStruct((NPAD, H), jnp.float32),
            jax.ShapeDtypeStruct((NPAD, H), jnp.bfloat16),
        ),
    )(aggp, g1, degp, W2, W2p, b1row)


def _tc3(aggp, g2, degp, b2row, brow, M1, mb1row, M2p, mb2row):
    def body(aggp_ref, g2_ref, degp_ref, b2_ref, brow_ref, m1_ref, mb1_ref,
             m2_ref, mb2_ref, out_ref):
        dis = _dis_col(degp_ref)
        h2 = jnp.maximum(
            (aggp_ref[0] + aggp_ref[1] + g2_ref[...]) * dis + b2_ref[...], 0.0)
        gids = lax.broadcasted_iota(jnp.int32, (B, NPAD), 0)
        oh = (brow_ref[...] == gids).astype(jnp.float32)
        ssum = jnp.dot(oh, h2, preferred_element_type=jnp.float32)
        cnt = jnp.dot(oh, jnp.ones((NPAD, 1), jnp.float32),
                      preferred_element_type=jnp.float32)
        pooled = ssum / jnp.maximum(cnt, 1.0)
        z = jnp.maximum(
            jnp.dot(pooled, m1_ref[...], preferred_element_type=jnp.float32)
            + mb1_ref[...], 0.0)
        out_ref[...] = (
            jnp.dot(z, m2_ref[...], preferred_element_type=jnp.float32)
            + mb2_ref[...])

    return pl.pallas_call(
        body,
        out_shape=jax.ShapeDtypeStruct((B, 128), jnp.float32),
    )(aggp, g2, degp, b2row, brow, M1, mb1row, M2p, mb2row)


@jax.jit
def kernel(x, ei, ew, b, W1, b1, W2, b2, M1, mb1, M2, mb2):
    npad_e = EPAD - E
    pidx = jnp.arange(npad_e, dtype=jnp.int32) % N
    src3 = jnp.concatenate([ei[0], pidx]).reshape(NW, C1, K1)
    dst3 = jnp.concatenate([ei[1], pidx]).reshape(NW, C1, K1)
    ew3 = jnp.concatenate(
        [ew, jnp.zeros((npad_e,), jnp.float32)]).reshape(NW, C1, K1)
    brow = jnp.pad(b, (0, NPAD - N), constant_values=B)[None, :]
    b1row = b1[None, :]
    b2row = b2[None, :]
    mb1row = mb1[None, :]
    M2p = jnp.pad(M2, ((0, 0), (0, 128 - M2.shape[1])))
    mb2row = jnp.pad(mb2, (0, 128 - mb2.shape[0]))[None, :]
    perm = jnp.asarray(PERM, dtype=jnp.int32)
    W1p = jnp.take(W1, perm, axis=1)
    W2p = jnp.take(W2, perm, axis=1)

    h, hp = _tc1a(x, W1, W1p)
    degp = _deg_kernel(dst3, ew3)

    g1, gb1 = _tc1b(h, hp, degp)
    agg1 = _agg_kernel(gb1, src3, dst3, ew3)
    g2, gb2 = _tc2(agg1, g1, degp, W2, W2p, b1row)
    agg2 = _agg_kernel(gb2, src3, dst3, ew3)
    out_full = _tc3(agg2, g2, degp, b2row, brow, M1, mb1row, M2p, mb2row)
    return out_full[:, :1]


# speedup vs baseline: 1.1511x; 1.0469x over previous
import functools

import jax
import jax.numpy as jnp
from jax import lax
from jax.experimental import pallas as pl
from jax.experimental.pallas import tpu as pltpu
from jax.experimental.pallas import tpu_sc as plsc

N = 10000
E = 320000
D = 128
H = 64
B = 32

NPAD = 10240
NC = 2
NS = 16
NW = NC * NS
K1 = 128
EPW = E // NW
C1 = EPW // K1
KR = EPW - C1 * K1
NBUF = 3
DBUF = 5
RPT = NPAD // NS

PERM = []
for _g in range(H // 32):
    for _k in range(16):
        PERM.append(32 * _g + _k)
        PERM.append(32 * _g + 16 + _k)


def _sc_mesh():
    return plsc.VectorSubcoreMesh(core_axis_name="c", subcore_axis_name="s")


_SC_PARAMS = pltpu.CompilerParams(use_tc_tiling_on_sc=False,
                                  needs_layout_passes=False)


def _deg_kernel(ei, ew):
    @functools.partial(
        pl.kernel,
        out_type=jax.ShapeDtypeStruct((NC, NPAD), jnp.float32),
        mesh=_sc_mesh(),
        compiler_params=_SC_PARAMS,
        scratch_types=(
            [pltpu.VMEM((K1,), jnp.int32)] * DBUF
            + [pltpu.VMEM((K1,), jnp.float32)] * DBUF
            + [pltpu.VMEM((KR,), jnp.int32),
               pltpu.VMEM((KR,), jnp.float32)]
            + [pltpu.VMEM((RPT,), jnp.float32),
               pltpu.VMEM_SHARED((NPAD,), jnp.float32)]
            + [pltpu.SemaphoreType.DMA] * (2 * DBUF)
        ),
    )
    def k(ei_h, ew_h, out_h, d0, d1, d2, d3, d4, e0, e1, e2, e3, e4,
          dr, er, zbuf, acc, *sems):
        dv = [d0, d1, d2, d3, d4]
        ev = [e0, e1, e2, e3, e4]
        sm = list(sems[:DBUF])
        ss = list(sems[DBUF:])
        cid = lax.axis_index("c")
        sid = lax.axis_index("s")
        wid = cid * NS + sid

        for i in range(RPT // 16):
            zbuf[pl.ds(i * 16, 16)] = jnp.zeros((16,), jnp.float32)
        pltpu.sync_copy(zbuf, acc.at[pl.ds(sid * RPT, RPT)])
        plsc.subcore_barrier()

        base = wid * EPW

        def issue_meta(c, s):
            pltpu.async_copy(ei_h.at[1, pl.ds(base + c * K1, K1)],
                             dv[s], sm[s])
            pltpu.async_copy(ew_h.at[pl.ds(base + c * K1, K1)], ev[s], sm[s])

        def wait_meta(c, s):
            pltpu.make_async_copy(ei_h.at[1, pl.ds(base + c * K1, K1)],
                                  dv[s], sm[s]).wait()
            pltpu.make_async_copy(ew_h.at[pl.ds(base + c * K1, K1)],
                                  ev[s], sm[s]).wait()

        for c0 in range(3):
            issue_meta(c0, c0)

        def emit(c, b, sdescs, prefetch):
            wait_meta(c, b)
            sdescs[b] = pltpu.async_copy(ev[b], acc.at[dv[b]], ss[b],
                                         add=True)
            if b >= 2:
                sdescs[b - 2].wait()
            if prefetch:
                issue_meta(c + 3, (b + 3) % DBUF)
            return sdescs

        def body(gi, carry):
            sdescs = [None] * DBUF
            for b in range(DBUF):
                sdescs = emit(gi * DBUF + b, b, sdescs, True)
            sdescs[DBUF - 2].wait()
            sdescs[DBUF - 1].wait()
            return carry

        ntrip = (C1 - 3) // DBUF
        lax.fori_loop(0, ntrip, body, 0)
        descs = {}
        for c in range(ntrip * DBUF, C1):
            b = c % DBUF
            wait_meta(c, b)
            descs[c] = pltpu.async_copy(ev[b], acc.at[dv[b]], ss[b],
                                        add=True)
            if c - 2 >= ntrip * DBUF:
                descs[c - 2].wait()
            if c + 3 < C1:
                issue_meta(c + 3, (c + 3) % DBUF)
        descs[C1 - 2].wait()
        descs[C1 - 1].wait()

        pltpu.sync_copy(ei_h.at[1, pl.ds(base + C1 * K1, KR)], dr)
        pltpu.sync_copy(ew_h.at[pl.ds(base + C1 * K1, KR)], er)
        pltpu.sync_copy(er, acc.at[dr], add=True)

        plsc.subcore_barrier()
        pltpu.sync_copy(acc.at[pl.ds(sid * RPT, RPT)],
                        out_h.at[cid, pl.ds(sid * RPT, RPT)])

    return k(ei, ew)


def _agg_kernel(g, ei, ew):
    @functools.partial(
        pl.kernel,
        out_type=jax.ShapeDtypeStruct((NC, NPAD, H), jnp.float32),
        mesh=_sc_mesh(),
        compiler_params=_SC_PARAMS,
        scratch_types=(
            [pltpu.VMEM((K1,), jnp.int32)] * NBUF
            + [pltpu.VMEM((K1,), jnp.int32)] * NBUF
            + [pltpu.VMEM((K1,), jnp.float32)] * NBUF
            + [pltpu.VMEM((K1, H), jnp.bfloat16)] * NBUF
            + [pltpu.VMEM((K1, H), jnp.float32)] * NBUF
            + [pltpu.VMEM((KR,), jnp.int32), pltpu.VMEM((KR,), jnp.int32),
               pltpu.VMEM((KR,), jnp.float32),
               pltpu.VMEM((KR, H), jnp.bfloat16),
               pltpu.VMEM((KR, H), jnp.float32)]
            + [pltpu.VMEM((64, H), jnp.float32),
               pltpu.VMEM_SHARED((NPAD, H), jnp.float32)]
            + [pltpu.SemaphoreType.DMA] * (3 * NBUF)
        ),
    )
    def k(g_h, ei_h, ew_h, out_h,
          sv0, sv1, sv2, dv0, dv1, dv2, ev0, ev1, ev2,
          r0, r1, r2, f0, f1, f2, svr, dvr, evr, rr, fr,
          zbuf, acc, *sems):
        sv = [sv0, sv1, sv2]
        dv = [dv0, dv1, dv2]
        ev = [ev0, ev1, ev2]
        rows = [r0, r1, r2]
        fb = [f0, f1, f2]
        sm = list(sems[:NBUF])
        sg = list(sems[NBUF:2 * NBUF])
        ss = list(sems[2 * NBUF:])
        cid = lax.axis_index("c")
        sid = lax.axis_index("s")
        wid = cid * NS + sid

        def zfill(i, carry):
            for j in range(H // 16):
                zbuf[i, pl.ds(j * 16, 16)] = jnp.zeros((16,), jnp.float32)
            return carry

        lax.fori_loop(0, 64, zfill, 0)

        def zcopy(i, carry):
            pltpu.sync_copy(zbuf, acc.at[pl.ds(sid * RPT + i * 64, 64)])
            return carry

        lax.fori_loop(0, RPT // 64, zcopy, 0)
        plsc.subcore_barrier()

        base = wid * EPW

        def issue_meta(c, s):
            pltpu.async_copy(ei_h.at[0, pl.ds(base + c * K1, K1)],
                             sv[s], sm[s])
            pltpu.async_copy(ei_h.at[1, pl.ds(base + c * K1, K1)],
                             dv[s], sm[s])
            pltpu.async_copy(ew_h.at[pl.ds(base + c * K1, K1)], ev[s], sm[s])

        def wait_meta(c, s):
            pltpu.make_async_copy(ei_h.at[0, pl.ds(base + c * K1, K1)],
                                  sv[s], sm[s]).wait()
            pltpu.make_async_copy(ei_h.at[1, pl.ds(base + c * K1, K1)],
                                  dv[s], sm[s]).wait()
            pltpu.make_async_copy(ew_h.at[pl.ds(base + c * K1, K1)],
                                  ev[s], sm[s]).wait()

        issue_meta(0, 0)
        issue_meta(1, 1)
        wait_meta(0, 0)
        pltpu.async_copy(g_h.at[sv[0]], rows[0], sg[0])

        def emit(c, b, sdescs, pf1, pf2):
            s, s1, s2 = b, (b + 1) % NBUF, (b + 2) % NBUF
            pltpu.make_async_copy(g_h.at[sv[s]], rows[s], sg[s]).wait()
            if pf1:
                wait_meta(c + 1, s1)
                pltpu.async_copy(g_h.at[sv[s1]], rows[s1], sg[s1])
            rs = rows[s]
            fs = fb[s]

            def scale(gg, c2):
                ew16 = ev[s][pl.ds(gg * 16, 16)]
                for j in range(16):
                    e = gg * 16 + j
                    w = ew16[j]
                    for f in range(H // 32):
                        x32 = rs[e, pl.ds(f * 32, 32)]
                        lo, hi = plsc.unpack(
                            x32, format=plsc.PackFormat.INTERLEAVED)
                        fs[e, pl.ds(f * 32, 16)] = lo * w
                        fs[e, pl.ds(f * 32 + 16, 16)] = hi * w
                return c2

            lax.fori_loop(0, K1 // 16, scale, 0)

            sdescs[b] = pltpu.async_copy(fb[s], acc.at[dv[s]], ss[s],
                                         add=True)
            if b >= 1:
                sdescs[b - 1].wait()
            if pf2:
                issue_meta(c + 2, s2)
            return sdescs

        def body(gi, carry):
            sdescs = [None] * NBUF
            for b in range(NBUF):
                sdescs = emit(gi * NBUF + b, b, sdescs, True, True)
            sdescs[NBUF - 1].wait()
            return carry

        lax.fori_loop(0, (C1 - 3) // NBUF, body, 0)
        sdescs = [None] * NBUF
        sdescs = emit(C1 - 3, 0, sdescs, True, True)
        sdescs = emit(C1 - 2, 1, sdescs, True, False)
        sdescs = emit(C1 - 1, 2, sdescs, False, False)
        sdescs[2].wait()

        rbase = base + C1 * K1
        pltpu.sync_copy(ei_h.at[0, pl.ds(rbase, KR)], svr)
        pltpu.sync_copy(ei_h.at[1, pl.ds(rbase, KR)], dvr)
        pltpu.sync_copy(ew_h.at[pl.ds(rbase, KR)], evr)
        pltpu.sync_copy(g_h.at[svr], rr)
        ew16 = evr[pl.ds(0, 16)]
        for j in range(KR):
            w = ew16[j]
            for f in range(H // 32):
                x32 = rr[j, pl.ds(f * 32, 32)]
                lo, hi = plsc.unpack(x32, format=plsc.PackFormat.INTERLEAVED)
                fr[j, pl.ds(f * 32, 16)] = lo * w
                fr[j, pl.ds(f * 32 + 16, 16)] = hi * w
        pltpu.sync_copy(fr, acc.at[dvr], add=True)

        plsc.subcore_barrier()
        pltpu.sync_copy(acc.at[pl.ds(sid * RPT, RPT)],
                        out_h.at[cid, pl.ds(sid * RPT, RPT)])

    return k(g, ei, ew)


def _tc1a(x, W1, W1p):
    def body(x_ref, w_ref, wp_ref, h_ref, hp_ref):
        x = x_ref[...]
        zp = jnp.zeros((NPAD - N, H), jnp.float32)
        h_ref[...] = jnp.concatenate(
            [jnp.dot(x, w_ref[...], preferred_element_type=jnp.float32), zp],
            axis=0)
        hp_ref[...] = jnp.concatenate(
            [jnp.dot(x, wp_ref[...], preferred_element_type=jnp.float32), zp],
            axis=0)

    return pl.pallas_call(
        body,
        out_shape=(
            jax.ShapeDtypeStruct((NPAD, H), jnp.float32),
            jax.ShapeDtypeStruct((NPAD, H), jnp.float32),
        ),
    )(x, W1, W1p)


def _dis_col(degp_ref):
    deg = degp_ref[0] + degp_ref[1] + 1.0
    return lax.rsqrt(deg).reshape(NPAD, 1)


def _tc1b(h, hp, degp):
    def body(h_ref, hp_ref, degp_ref, g_ref, gb_ref):
        dis = _dis_col(degp_ref)
        g_ref[...] = h_ref[...] * dis
        gb_ref[...] = (hp_ref[...] * dis).astype(jnp.bfloat16)

    return pl.pallas_call(
        body,
        out_shape=(
            jax.ShapeDtypeStruct((NPAD, H), jnp.float32),
            jax.ShapeDtypeStruct((NPAD, H), jnp.bfloat16),
        ),
    )(h, hp, degp)


def _tc2(aggp, g1, degp, W2, W2p, b1row):
    def body(aggp_ref, g1_ref, degp_ref, w_ref, wp_ref, b_ref, g2_ref, gb_ref):
        dis = _dis_col(degp_ref)
        h = jnp.maximum(
            (aggp_ref[0] + aggp_ref[1] + g1_ref[...]) * dis + b_ref[...], 0.0)
        t = jnp.dot(h, w_ref[...], preferred_element_type=jnp.float32)
        tp = jnp.dot(h, wp_ref[...], preferred_element_type=jnp.float32)
        g2_ref[...] = t * dis
        gb_ref[...] = (tp * dis).astype(jnp.bfloat16)

    return pl.pallas_call(
        body,
        out_shape=(
            jax.ShapeDtypeStruct((NPAD, H), jnp.float32),
            jax.ShapeDtypeStruct((NPAD, H), jnp.bfloat16),
        ),
    )(aggp, g1, degp, W2, W2p, b1row)


def _tc3(aggp, g2, degp, b2row, brow, M1, mb1row, M2p, mb2row):
    def body(aggp_ref, g2_ref, degp_ref, b2_ref, brow_ref, m1_ref, mb1_ref,
             m2_ref, mb2_ref, out_ref):
        dis = _dis_col(degp_ref)
        h2 = jnp.maximum(
            (aggp_ref[0] + aggp_ref[1] + g2_ref[...]) * dis + b2_ref[...], 0.0)
        gids = lax.broadcasted_iota(jnp.int32, (B, NPAD), 0)
        oh = (brow_ref[...] == gids).astype(jnp.float32)
        ssum = jnp.dot(oh, h2, preferred_element_type=jnp.float32)
        cnt = jnp.dot(oh, jnp.ones((NPAD, 1), jnp.float32),
                      preferred_element_type=jnp.float32)
        pooled = ssum / jnp.maximum(cnt, 1.0)
        z = jnp.maximum(
            jnp.dot(pooled, m1_ref[...], preferred_element_type=jnp.float32)
            + mb1_ref[...], 0.0)
        out_ref[...] = (
            jnp.dot(z, m2_ref[...], preferred_element_type=jnp.float32)
            + mb2_ref[...])

    return pl.pallas_call(
        body,
        out_shape=jax.ShapeDtypeStruct((B, 128), jnp.float32),
    )(aggp, g2, degp, b2row, brow, M1, mb1row, M2p, mb2row)


@jax.jit
def kernel(x, ei, ew, b, W1, b1, W2, b2, M1, mb1, M2, mb2):
    brow = jnp.pad(b, (0, NPAD - N), constant_values=B)[None, :]
    b1row = b1[None, :]
    b2row = b2[None, :]
    mb1row = mb1[None, :]
    M2p = jnp.pad(M2, ((0, 0), (0, 128 - M2.shape[1])))
    mb2row = jnp.pad(mb2, (0, 128 - mb2.shape[0]))[None, :]
    perm = jnp.asarray(PERM, dtype=jnp.int32)
    W1p = jnp.take(W1, perm, axis=1)
    W2p = jnp.take(W2, perm, axis=1)

    h, hp = _tc1a(x, W1, W1p)
    degp = _deg_kernel(ei, ew)

    g1, gb1 = _tc1b(h, hp, degp)
    agg1 = _agg_kernel(gb1, ei, ew)
    g2, gb2 = _tc2(agg1, g1, degp, W2, W2p, b1row)
    agg2 = _agg_kernel(gb2, ei, ew)
    out_full = _tc3(agg2, g2, degp, b2row, brow, M1, mb1row, M2p, mb2row)
    return out_full[:, :1]
